# Initial kernel scaffold; baseline (speedup 1.0000x reference)
import jax
import jax.numpy as jnp
from jax.experimental import pallas as pl


def kernel(x, tables):
    raise NotImplementedError("write your pallas kernel here")


# speedup vs baseline: 1.1951x; 1.1951x over previous
import jax
import jax.numpy as jnp
from jax import lax
from jax.experimental import pallas as pl
from jax.experimental.pallas import tpu as pltpu
from jax.experimental.pallas import tpu_sc as plsc

_BATCH = 16384
_NUM_FIELDS = 26
_VOCAB = 100000
_EMB_DIM = 32

_NC = 2
_NS = 16
_NW = _NC * _NS
_BPW = _BATCH // _NW
_CHUNK = 128
_KPF = _BPW // _CHUNK
_IDX_ROWS = _NUM_FIELDS * _KPF


def _sc_body(idx_hbm, table_hbm, out_hbm, idx_v, acc, buf0, buf1,
             sem_a, sem0, sem1):
    w = lax.axis_index("s") * _NC + lax.axis_index("c")
    base = w * _BPW
    pltpu.sync_copy(idx_hbm.at[w], idx_v)

    def fire(field, dst, sem):
        for k in range(_KPF):
            pltpu.async_copy(
                table_hbm.at[idx_v.at[field * _KPF + k]],
                dst.at[pl.ds(k * _CHUNK, _CHUNK)],
                sem)

    def drain(dst, sem):
        pltpu.make_async_copy(table_hbm.at[pl.ds(0, _BPW)], dst, sem).wait()

    def accum(buf):
        def body(blk, carry):
            for r in range(8):
                b = blk * 8 + r
                for h in range(2):
                    sl = pl.ds(h * 16, 16)
                    plsc.addupdate(acc.at[b, sl], buf[b, sl])
            return carry
        lax.fori_loop(0, _BPW // 8, body, 0)

    fire(0, acc, sem_a)
    fire(1, buf1, sem1)
    drain(acc, sem_a)

    def loop_body(t, carry):
        fe = 2 + 2 * t
        fire(fe, buf0, sem0)
        drain(buf1, sem1)
        accum(buf1)
        fire(fe + 1, buf1, sem1)
        drain(buf0, sem0)
        accum(buf0)
        return carry

    lax.fori_loop(0, (_NUM_FIELDS - 2) // 2, loop_body, 0)
    drain(buf1, sem1)
    accum(buf1)
    pltpu.sync_copy(acc, out_hbm.at[pl.ds(base, _BPW)])


def kernel(x, tables):
    offs = (jnp.arange(_NUM_FIELDS, dtype=jnp.int32) * _VOCAB)[None, :]
    flat = x + offs
    idx_arr = (flat.T.reshape(_NUM_FIELDS, _NW, _BPW)
               .transpose(1, 0, 2)
               .reshape(_NW, _IDX_ROWS, _CHUNK))
    table_flat = tables.reshape(_NUM_FIELDS * _VOCAB, _EMB_DIM)

    f = pl.kernel(
        _sc_body,
        out_type=jax.ShapeDtypeStruct((_BATCH, _EMB_DIM), jnp.float32),
        mesh=plsc.VectorSubcoreMesh(core_axis_name="c", subcore_axis_name="s"),
        scratch_types=[
            pltpu.VMEM((_IDX_ROWS, _CHUNK), jnp.int32),
            pltpu.VMEM((_BPW, _EMB_DIM), jnp.float32),
            pltpu.VMEM((_BPW, _EMB_DIM), jnp.float32),
            pltpu.VMEM((_BPW, _EMB_DIM), jnp.float32),
            pltpu.SemaphoreType.DMA,
            pltpu.SemaphoreType.DMA,
            pltpu.SemaphoreType.DMA,
        ],
        compiler_params=pltpu.CompilerParams(use_tc_tiling_on_sc=False),
    )
    return f(idx_arr, table_flat)


# speedup vs baseline: 3.7903x; 3.1715x over previous
import jax
import jax.numpy as jnp
from jax import lax
from jax.experimental import pallas as pl
from jax.experimental.pallas import tpu as pltpu
from jax.experimental.pallas import tpu_sc as plsc

_BATCH = 16384
_NUM_FIELDS = 26
_VOCAB = 100000
_EMB_DIM = 32

_NC = 2
_NS = 16
_NW = _NC * _NS
_HB = _BATCH // 2


def _sc_body(x_hbm, table_hbm, out_hbm, row_v, idx_v, acc, sem_r, sem_i):
    s = lax.axis_index("s") * _NC + lax.axis_index("c")

    def field(i, first):
        row_cp = pltpu.async_copy(table_hbm.at[i, s, :], row_v, sem_r)
        idx_cp = pltpu.async_copy(x_hbm.at[i, pl.ds(0, _HB)], idx_v, sem_i)
        row_cp.wait()

        def process(half_base):
            def chunk(j, carry):
                v = idx_v[pl.ds(j * 16, 16)]
                g = plsc.load_gather(row_v, [v])
                sl = pl.ds(half_base + j * 16, 16)
                if first:
                    acc[sl] = g
                else:
                    plsc.addupdate(acc.at[sl], g)
                return carry
            lax.fori_loop(0, _HB // 16, chunk, 0)

        idx_cp.wait()
        process(0)
        idx_cp2 = pltpu.async_copy(x_hbm.at[i, pl.ds(_HB, _HB)], idx_v, sem_i)
        idx_cp2.wait()
        process(_HB)

    field(0, True)
    lax.fori_loop(1, _NUM_FIELDS, lambda i, c: (field(i, False), c)[1], 0)
    pltpu.sync_copy(acc, out_hbm.at[s])


def kernel(x, tables):
    x_t = x.T
    tables_t = tables.transpose(0, 2, 1)

    f = pl.kernel(
        _sc_body,
        out_type=jax.ShapeDtypeStruct((_EMB_DIM, _BATCH), jnp.float32),
        mesh=plsc.VectorSubcoreMesh(core_axis_name="c", subcore_axis_name="s"),
        scratch_types=[
            pltpu.VMEM((_VOCAB,), jnp.float32),
            pltpu.VMEM((_HB,), jnp.int32),
            pltpu.VMEM((_BATCH,), jnp.float32),
            pltpu.SemaphoreType.DMA,
            pltpu.SemaphoreType.DMA,
        ],
        compiler_params=pltpu.CompilerParams(
            use_tc_tiling_on_sc=True, needs_layout_passes=False),
    )
    out_t = f(x_t, tables_t)
    return out_t.T


# speedup vs baseline: 4.7760x; 1.2600x over previous
import jax
import jax.numpy as jnp
from jax import lax
from jax.experimental import pallas as pl
from jax.experimental.pallas import tpu as pltpu
from jax.experimental.pallas import tpu_sc as plsc

_BATCH = 16384
_NUM_FIELDS = 26
_VOCAB = 100000
_EMB_DIM = 32

_NC = 2
_NS = 16
_NW = _NC * _NS
_HB = _BATCH // 2


def _sc_body(x_hbm, table_hbm, out_hbm, row_v, idx_v, acc, sem_r, sem_i):
    s = lax.axis_index("s") * _NC + lax.axis_index("c")

    def field(i, first):
        row_cp = pltpu.async_copy(table_hbm.at[i, s, :], row_v, sem_r)
        idx_cp = pltpu.async_copy(x_hbm.at[i, pl.ds(0, _HB)], idx_v, sem_i)
        row_cp.wait()

        def process(half_base):
            def chunk(j, carry):
                v = idx_v[pl.ds(j * 16, 16)]
                g = plsc.load_gather(row_v, [v])
                sl = pl.ds(half_base + j * 16, 16)
                if first:
                    acc[sl] = g
                else:
                    plsc.addupdate(acc.at[sl], g)
                return carry
            lax.fori_loop(0, _HB // 16, chunk, 0, unroll=8)

        idx_cp.wait()
        process(0)
        idx_cp2 = pltpu.async_copy(x_hbm.at[i, pl.ds(_HB, _HB)], idx_v, sem_i)
        idx_cp2.wait()
        process(_HB)

    field(0, True)
    lax.fori_loop(1, _NUM_FIELDS, lambda i, c: (field(i, False), c)[1], 0)
    pltpu.sync_copy(acc, out_hbm.at[s])


def kernel(x, tables):
    x_t = x.T
    tables_t = tables.transpose(0, 2, 1)

    f = pl.kernel(
        _sc_body,
        out_type=jax.ShapeDtypeStruct((_EMB_DIM, _BATCH), jnp.float32),
        mesh=plsc.VectorSubcoreMesh(core_axis_name="c", subcore_axis_name="s"),
        scratch_types=[
            pltpu.VMEM((_VOCAB,), jnp.float32),
            pltpu.VMEM((_HB,), jnp.int32),
            pltpu.VMEM((_BATCH,), jnp.float32),
            pltpu.SemaphoreType.DMA,
            pltpu.SemaphoreType.DMA,
        ],
        compiler_params=pltpu.CompilerParams(
            use_tc_tiling_on_sc=True, needs_layout_passes=False),
    )
    out_t = f(x_t, tables_t)
    return out_t.T
